# baseline (speedup 1.0000x reference)
import functools

import jax
import jax.numpy as jnp
from jax import lax
from jax.experimental import pallas as pl
from jax.experimental.pallas import tpu as pltpu
from jax.experimental.pallas import tpu_sc as plsc

N = 10000
E = 160000
NSUB = 16
CHUNK = 128
NCHUNK = 80
EPT = NCHUNK * CHUNK
PADE = NSUB * EPT
ROWS_PER = 632
NROWS = NSUB * ROWS_PER
W64 = 64
NSLOT = 8
LOOKA = 4
MT = 400
GRID_M = N // MT

f32 = jnp.float32
_mesh = plsc.VectorSubcoreMesh(core_axis_name="c", subcore_axis_name="s")


@functools.partial(
    pl.kernel,
    out_type=(jax.ShapeDtypeStruct((NROWS, W64), f32),) * 2,
    mesh=_mesh,
    scratch_types=[
        pltpu.VMEM((NCHUNK, CHUNK), jnp.int32),
        pltpu.VMEM((NCHUNK, CHUNK), jnp.int32),
        pltpu.VMEM((NSLOT, CHUNK, W64), f32),
        pltpu.VMEM_SHARED((NROWS, W64), f32),
        pltpu.SemaphoreType.DMA((NSLOT,)),
        pltpu.SemaphoreType.DMA((NSLOT,)),
    ],
    compiler_params=pltpu.CompilerParams(use_tc_tiling_on_sc=False),
)
def _seg2(tbl0, tbl1, srcs, dsts, zrows, out0, out1,
          sidx, didx, bufs, acc, gsems, ssems):
    c = lax.axis_index("c")
    s = lax.axis_index("s")
    row0 = s * ROWS_PER
    pltpu.sync_copy(srcs.at[c, s], sidx)
    pltpu.sync_copy(dsts.at[c, s], didx)

    def run(tbl, out):
        def gath(j, b):
            return pltpu.make_async_copy(tbl.at[sidx.at[j]], bufs.at[b],
                                         gsems.at[b])

        def scat(j, b):
            return pltpu.make_async_copy(bufs.at[b], acc.at[didx.at[j]],
                                         ssems.at[b])

        pltpu.sync_copy(zrows, acc.at[pl.ds(row0, ROWS_PER)])
        plsc.subcore_barrier()
        for b in range(LOOKA):
            gath(b, b).start()

        def body(gg, carry):
            j0 = gg * NSLOT
            for b in range(NSLOT):
                j = j0 + b
                gath(j, b).wait()
                pltpu.async_copy(bufs.at[b], acc.at[didx.at[j]],
                                 ssems.at[b], add=True)

                @pl.when(j + LOOKA < NCHUNK)
                def _():
                    bl = (b + LOOKA) % NSLOT

                    @pl.when(j >= NSLOT - LOOKA)
                    def _():
                        scat(j - (NSLOT - LOOKA), bl).wait()

                    gath(j + LOOKA, bl).start()
            return carry

        lax.fori_loop(0, NCHUNK // NSLOT, body, 0)
        for b in range(NSLOT):
            scat(NCHUNK - NSLOT + b, b).wait()
        plsc.subcore_barrier()
        pltpu.sync_copy(acc.at[pl.ds(row0, ROWS_PER)],
                        out.at[pl.ds(row0, ROWS_PER)])

    @pl.when(c == 0)
    def _():
        run(tbl0, out0)

    @pl.when(c == 1)
    def _():
        run(tbl1, out1)


@functools.partial(
    pl.kernel,
    out_type=(jax.ShapeDtypeStruct((NROWS, 16), f32),
              jax.ShapeDtypeStruct((NROWS, 16), f32)),
    mesh=_mesh,
    scratch_types=[
        pltpu.VMEM((NCHUNK, CHUNK), jnp.int32),
        pltpu.VMEM((CHUNK, 16), f32),
        pltpu.VMEM_SHARED((NROWS, 16), f32),
    ],
    compiler_params=pltpu.CompilerParams(use_tc_tiling_on_sc=False),
)
def _deg2(dsts, ones_rows, zrows, out0, out1, didx, onesv, acc):
    c = lax.axis_index("c")
    s = lax.axis_index("s")
    row0 = s * ROWS_PER
    pltpu.sync_copy(dsts.at[c, s], didx)
    pltpu.sync_copy(ones_rows, onesv)
    pltpu.sync_copy(zrows, acc.at[pl.ds(row0, ROWS_PER)])
    plsc.subcore_barrier()

    def body(j, carry):
        pltpu.sync_copy(onesv, acc.at[didx.at[j]], add=True)
        return carry

    lax.fori_loop(0, NCHUNK, body, 0)
    plsc.subcore_barrier()

    @pl.when(c == 0)
    def _():
        pltpu.sync_copy(acc.at[pl.ds(row0, ROWS_PER)],
                        out0.at[pl.ds(row0, ROWS_PER)])

    @pl.when(c == 1)
    def _():
        pltpu.sync_copy(acc.at[pl.ds(row0, ROWS_PER)],
                        out1.at[pl.ds(row0, ROWS_PER)])


def _prep(src, dst):
    pad = PADE - E
    src = jnp.concatenate([src.astype(jnp.int32),
                           jnp.zeros((pad,), jnp.int32)])
    dst = jnp.concatenate([dst.astype(jnp.int32),
                           jnp.full((pad,), N, jnp.int32)])
    return src.reshape(NSUB, NCHUNK, CHUNK), dst.reshape(NSUB, NCHUNK, CHUNK)


def _dense_combine(mats, scaled_by, degs, Ws, b, act, out_widths):
    OUT = Ws[0].shape[1]
    nm = len(mats)
    nd = len(degs)

    def body(*refs):
        mrefs = refs[:nm]
        drefs = refs[nm:nm + nd]
        wrefs = refs[nm + nd:nm + nd + nm]
        bref = refs[nm + nd + nm]
        orefs = refs[nm + nd + nm + 1:]
        rs = [1.0 / jnp.maximum(dr[:, 0:1], 1.0) for dr in
              [d[...] for d in drefs]]
        res = jnp.zeros((MT, OUT), f32)
        for mref, sb, wref in zip(mrefs, scaled_by, wrefs):
            xm = mref[...]
            if sb is not None:
                xm = xm * rs[sb]
            res = res + jnp.dot(xm, wref[...], preferred_element_type=f32)
        res = res + bref[...]
        if act:
            res = jnp.maximum(res, 0.0)
        off = 0
        for oref, w in zip(orefs, out_widths):
            oref[...] = res[:, off:off + w]
            off += w

    in_specs = (
        [pl.BlockSpec((MT, m.shape[1]), lambda i: (i, 0)) for m in mats]
        + [pl.BlockSpec((MT, 16), lambda i: (i, 0)) for _ in degs]
        + [pl.BlockSpec(w.shape, lambda i: (0, 0)) for w in Ws]
        + [pl.BlockSpec((1, OUT), lambda i: (0, 0))]
    )
    out_shape = tuple(jax.ShapeDtypeStruct((N, w), f32) for w in out_widths)
    out_specs = tuple(pl.BlockSpec((MT, w), lambda i: (i, 0))
                      for w in out_widths)
    res = pl.pallas_call(
        body, grid=(GRID_M,), in_specs=in_specs, out_specs=out_specs,
        out_shape=out_shape,
    )(*mats, *degs, *Ws, b.reshape(1, OUT))
    return res


def _premm(hmats, W0s, W1s, Wls, b):
    nh = len(hmats)

    def body(*refs):
        hrefs = refs[:nh]
        w0refs = refs[nh:2 * nh]
        w1refs = refs[2 * nh:3 * nh]
        wlrefs = refs[3 * nh:4 * nh]
        bref = refs[4 * nh]
        o0a, o0b, o1a, o1b, o2 = refs[4 * nh + 1:]
        hs = [h[...] for h in hrefs]
        r0 = sum(jnp.dot(h, w[...], preferred_element_type=f32)
                 for h, w in zip(hs, w0refs))
        r1 = sum(jnp.dot(h, w[...], preferred_element_type=f32)
                 for h, w in zip(hs, w1refs))
        r2 = sum(jnp.dot(h, w[...], preferred_element_type=f32)
                 for h, w in zip(hs, wlrefs)) + bref[...]
        o0a[...] = r0[:, :W64]
        o0b[...] = r0[:, W64:]
        o1a[...] = r1[:, :W64]
        o1b[...] = r1[:, W64:]
        o2[...] = r2

    in_specs = (
        [pl.BlockSpec((MT, W64), lambda i: (i, 0)) for _ in hmats]
        + [pl.BlockSpec(w.shape, lambda i: (0, 0))
           for w in (*W0s, *W1s, *Wls)]
        + [pl.BlockSpec((1, 128), lambda i: (0, 0))]
    )
    out_shape = (tuple(jax.ShapeDtypeStruct((N, W64), f32) for _ in range(4))
                 + (jax.ShapeDtypeStruct((N, 128), f32),))
    out_specs = (tuple(pl.BlockSpec((MT, W64), lambda i: (i, 0))
                       for _ in range(4))
                 + (pl.BlockSpec((MT, 128), lambda i: (i, 0)),))
    return pl.pallas_call(
        body, grid=(GRID_M,), in_specs=in_specs, out_specs=out_specs,
        out_shape=out_shape,
    )(*hmats, *W0s, *W1s, *Wls, b.reshape(1, 128))


def _final(zpieces, deg0, deg1, y2):
    def body(z0a, z0b, z1a, z1b, d0r, d1r, y2r, o):
        r0 = 1.0 / jnp.maximum(d0r[:, 0:1], 1.0)
        r1 = 1.0 / jnp.maximum(d1r[:, 0:1], 1.0)
        z0 = jnp.concatenate([z0a[...], z0b[...]], axis=1)
        z1 = jnp.concatenate([z1a[...], z1b[...]], axis=1)
        o[...] = y2r[...] + z0 * r0 + z1 * r1

    in_specs = (
        [pl.BlockSpec((MT, W64), lambda i: (i, 0))] * 4
        + [pl.BlockSpec((MT, 16), lambda i: (i, 0))] * 2
        + [pl.BlockSpec((MT, 128), lambda i: (i, 0))]
    )
    return pl.pallas_call(
        body, grid=(GRID_M,), in_specs=in_specs,
        out_specs=pl.BlockSpec((MT, 128), lambda i: (i, 0)),
        out_shape=jax.ShapeDtypeStruct((N, 128), f32),
    )(*zpieces, deg0, deg1, y2)


def kernel(x, rel0_src, rel0_dst, rel1_src, rel1_dst,
           W0_r0, W0_r1, Wl0, b0, W1_r0, W1_r1, Wl1, b1,
           W2_r0, W2_r1, Wl2, b2):
    s0, d0 = _prep(rel0_src, rel0_dst)
    s1, d1 = _prep(rel1_src, rel1_dst)
    srcs = jnp.stack([s0, s1])
    dsts = jnp.stack([d0, d1])
    zrows = jnp.zeros((ROWS_PER, W64), f32)
    zrows16 = jnp.zeros((ROWS_PER, 16), f32)
    ones16 = jnp.ones((CHUNK, 16), f32)

    deg0, deg1 = _deg2(dsts, ones16, zrows16)

    xa, xb = x[:, :W64], x[:, W64:]
    a0a, a1a = _seg2(xa, xa, srcs, dsts, zrows)
    a0b, a1b = _seg2(xb, xb, srcs, dsts, zrows)
    h1 = _dense_combine(
        [a0a, a0b, a1a, a1b, x], [0, 0, 1, 1, None], [deg0, deg1],
        [W0_r0[:W64], W0_r0[W64:], W0_r1[:W64], W0_r1[W64:], Wl0],
        b0, True, (W64,) * 4)
    h10, h11, h12, h13 = h1

    b0a, b1a = _seg2(h10, h10, srcs, dsts, zrows)
    b0b, b1b = _seg2(h11, h11, srcs, dsts, zrows)
    b0c, b1c = _seg2(h12, h12, srcs, dsts, zrows)
    b0d, b1d = _seg2(h13, h13, srcs, dsts, zrows)
    h2 = _dense_combine(
        [b0a, b0b, b0c, b0d, b1a, b1b, b1c, b1d, h10, h11, h12, h13],
        [0, 0, 0, 0, 1, 1, 1, 1, None, None, None, None],
        [deg0, deg1],
        [W1_r0[:W64], W1_r0[W64:128], W1_r0[128:192], W1_r0[192:],
         W1_r1[:W64], W1_r1[W64:128], W1_r1[128:192], W1_r1[192:],
         Wl1[:W64], Wl1[W64:128], Wl1[128:192], Wl1[192:]],
        b1, True, (W64,) * 4)
    h20, h21, h22, h23 = h2

    hmats = [h20, h21, h22, h23]
    wrows = [(0, W64), (W64, 128), (128, 192), (192, 256)]
    y0a, y0b, y1a, y1b, y2 = _premm(
        hmats,
        [W2_r0[a:bb] for a, bb in wrows],
        [W2_r1[a:bb] for a, bb in wrows],
        [Wl2[a:bb] for a, bb in wrows],
        b2)
    z0a, z1a = _seg2(y0a, y1a, srcs, dsts, zrows)
    z0b, z1b = _seg2(y0b, y1b, srcs, dsts, zrows)
    return _final([z0a, z0b, z1a, z1b], deg0, deg1, y2)


# speedup vs baseline: 1.9909x; 1.9909x over previous
import functools

import jax
import jax.numpy as jnp
from jax import lax
from jax.experimental import pallas as pl
from jax.experimental.pallas import tpu as pltpu
from jax.experimental.pallas import tpu_sc as plsc

N = 10000
E = 160000
NSUB = 16
CHUNK = 128
NCHUNK = 80
EPT = NCHUNK * CHUNK
PADE = NSUB * EPT
ROWS_PER = 632
NROWS = NSUB * ROWS_PER
W64 = 64
NSLOT = 2
LOOKA = 1
TROWS = N // NSUB
MT = 400
GRID_M = N // MT

f32 = jnp.float32
_mesh = plsc.VectorSubcoreMesh(core_axis_name="c", subcore_axis_name="s")


@functools.partial(
    pl.kernel,
    out_type=(jax.ShapeDtypeStruct((NROWS, W64), f32),) * 2,
    mesh=_mesh,
    scratch_types=[
        pltpu.VMEM((NCHUNK, CHUNK), jnp.int32),
        pltpu.VMEM((NCHUNK, CHUNK), jnp.int32),
        pltpu.VMEM((NSLOT, CHUNK, W64), f32),
        pltpu.VMEM_SHARED((NROWS, W64), f32),
        pltpu.VMEM_SHARED((N, W64), f32),
        pltpu.SemaphoreType.DMA((NSLOT,)),
        pltpu.SemaphoreType.DMA((NSLOT,)),
    ],
    compiler_params=pltpu.CompilerParams(use_tc_tiling_on_sc=False),
)
def _seg2(tbl0, tbl1, srcs, dsts, zrows, out0, out1,
          sidx, didx, bufs, acc, tsp, gsems, ssems):
    c = lax.axis_index("c")
    s = lax.axis_index("s")
    row0 = s * ROWS_PER
    pltpu.sync_copy(srcs.at[c, s], sidx)
    pltpu.sync_copy(dsts.at[c, s], didx)

    def run(tbl, out):
        def gath(j, b):
            return pltpu.make_async_copy(tsp.at[sidx.at[j]], bufs.at[b],
                                         gsems.at[b])

        def scat(j, b):
            return pltpu.make_async_copy(bufs.at[b], acc.at[didx.at[j]],
                                         ssems.at[b])

        pltpu.sync_copy(zrows, acc.at[pl.ds(row0, ROWS_PER)])
        pltpu.sync_copy(tbl.at[pl.ds(s * TROWS, TROWS)],
                        tsp.at[pl.ds(s * TROWS, TROWS)])
        plsc.subcore_barrier()
        for b in range(LOOKA):
            gath(b, b).start()

        def body(gg, carry):
            j0 = gg * NSLOT
            for b in range(NSLOT):
                j = j0 + b
                gath(j, b).wait()
                pltpu.async_copy(bufs.at[b], acc.at[didx.at[j]],
                                 ssems.at[b], add=True)

                @pl.when(j + LOOKA < NCHUNK)
                def _():
                    bl = (b + LOOKA) % NSLOT

                    @pl.when(j >= NSLOT - LOOKA)
                    def _():
                        scat(j - (NSLOT - LOOKA), bl).wait()

                    gath(j + LOOKA, bl).start()
            return carry

        lax.fori_loop(0, NCHUNK // NSLOT, body, 0)
        for b in range(NSLOT):
            scat(NCHUNK - NSLOT + b, b).wait()
        plsc.subcore_barrier()
        pltpu.sync_copy(acc.at[pl.ds(row0, ROWS_PER)],
                        out.at[pl.ds(row0, ROWS_PER)])

    @pl.when(c == 0)
    def _():
        run(tbl0, out0)

    @pl.when(c == 1)
    def _():
        run(tbl1, out1)


@functools.partial(
    pl.kernel,
    out_type=(jax.ShapeDtypeStruct((NROWS, 16), f32),
              jax.ShapeDtypeStruct((NROWS, 16), f32)),
    mesh=_mesh,
    scratch_types=[
        pltpu.VMEM((NCHUNK, CHUNK), jnp.int32),
        pltpu.VMEM((CHUNK, 16), f32),
        pltpu.VMEM_SHARED((NROWS, 16), f32),
    ],
    compiler_params=pltpu.CompilerParams(use_tc_tiling_on_sc=False),
)
def _deg2(dsts, ones_rows, zrows, out0, out1, didx, onesv, acc):
    c = lax.axis_index("c")
    s = lax.axis_index("s")
    row0 = s * ROWS_PER
    pltpu.sync_copy(dsts.at[c, s], didx)
    pltpu.sync_copy(ones_rows, onesv)
    pltpu.sync_copy(zrows, acc.at[pl.ds(row0, ROWS_PER)])
    plsc.subcore_barrier()

    def body(j, carry):
        pltpu.sync_copy(onesv, acc.at[didx.at[j]], add=True)
        return carry

    lax.fori_loop(0, NCHUNK, body, 0)
    plsc.subcore_barrier()

    @pl.when(c == 0)
    def _():
        pltpu.sync_copy(acc.at[pl.ds(row0, ROWS_PER)],
                        out0.at[pl.ds(row0, ROWS_PER)])

    @pl.when(c == 1)
    def _():
        pltpu.sync_copy(acc.at[pl.ds(row0, ROWS_PER)],
                        out1.at[pl.ds(row0, ROWS_PER)])


def _prep(src, dst):
    pad = PADE - E
    src = jnp.concatenate([src.astype(jnp.int32),
                           jnp.zeros((pad,), jnp.int32)])
    dst = jnp.concatenate([dst.astype(jnp.int32),
                           jnp.full((pad,), N, jnp.int32)])
    return src.reshape(NSUB, NCHUNK, CHUNK), dst.reshape(NSUB, NCHUNK, CHUNK)


def _dense_combine(mats, scaled_by, degs, Ws, b, act, out_widths):
    OUT = Ws[0].shape[1]
    nm = len(mats)
    nd = len(degs)

    def body(*refs):
        mrefs = refs[:nm]
        drefs = refs[nm:nm + nd]
        wrefs = refs[nm + nd:nm + nd + nm]
        bref = refs[nm + nd + nm]
        orefs = refs[nm + nd + nm + 1:]
        rs = [1.0 / jnp.maximum(dr[:, 0:1], 1.0) for dr in
              [d[...] for d in drefs]]
        res = jnp.zeros((MT, OUT), f32)
        for mref, sb, wref in zip(mrefs, scaled_by, wrefs):
            xm = mref[...]
            if sb is not None:
                xm = xm * rs[sb]
            res = res + jnp.dot(xm, wref[...], preferred_element_type=f32)
        res = res + bref[...]
        if act:
            res = jnp.maximum(res, 0.0)
        off = 0
        for oref, w in zip(orefs, out_widths):
            oref[...] = res[:, off:off + w]
            off += w

    in_specs = (
        [pl.BlockSpec((MT, m.shape[1]), lambda i: (i, 0)) for m in mats]
        + [pl.BlockSpec((MT, 16), lambda i: (i, 0)) for _ in degs]
        + [pl.BlockSpec(w.shape, lambda i: (0, 0)) for w in Ws]
        + [pl.BlockSpec((1, OUT), lambda i: (0, 0))]
    )
    out_shape = tuple(jax.ShapeDtypeStruct((N, w), f32) for w in out_widths)
    out_specs = tuple(pl.BlockSpec((MT, w), lambda i: (i, 0))
                      for w in out_widths)
    res = pl.pallas_call(
        body, grid=(GRID_M,), in_specs=in_specs, out_specs=out_specs,
        out_shape=out_shape,
    )(*mats, *degs, *Ws, b.reshape(1, OUT))
    return res


def _premm(hmats, W0s, W1s, Wls, b):
    nh = len(hmats)

    def body(*refs):
        hrefs = refs[:nh]
        w0refs = refs[nh:2 * nh]
        w1refs = refs[2 * nh:3 * nh]
        wlrefs = refs[3 * nh:4 * nh]
        bref = refs[4 * nh]
        o0a, o0b, o1a, o1b, o2 = refs[4 * nh + 1:]
        hs = [h[...] for h in hrefs]
        r0 = sum(jnp.dot(h, w[...], preferred_element_type=f32)
                 for h, w in zip(hs, w0refs))
        r1 = sum(jnp.dot(h, w[...], preferred_element_type=f32)
                 for h, w in zip(hs, w1refs))
        r2 = sum(jnp.dot(h, w[...], preferred_element_type=f32)
                 for h, w in zip(hs, wlrefs)) + bref[...]
        o0a[...] = r0[:, :W64]
        o0b[...] = r0[:, W64:]
        o1a[...] = r1[:, :W64]
        o1b[...] = r1[:, W64:]
        o2[...] = r2

    in_specs = (
        [pl.BlockSpec((MT, W64), lambda i: (i, 0)) for _ in hmats]
        + [pl.BlockSpec(w.shape, lambda i: (0, 0))
           for w in (*W0s, *W1s, *Wls)]
        + [pl.BlockSpec((1, 128), lambda i: (0, 0))]
    )
    out_shape = (tuple(jax.ShapeDtypeStruct((N, W64), f32) for _ in range(4))
                 + (jax.ShapeDtypeStruct((N, 128), f32),))
    out_specs = (tuple(pl.BlockSpec((MT, W64), lambda i: (i, 0))
                       for _ in range(4))
                 + (pl.BlockSpec((MT, 128), lambda i: (i, 0)),))
    return pl.pallas_call(
        body, grid=(GRID_M,), in_specs=in_specs, out_specs=out_specs,
        out_shape=out_shape,
    )(*hmats, *W0s, *W1s, *Wls, b.reshape(1, 128))


def _final(zpieces, deg0, deg1, y2):
    def body(z0a, z0b, z1a, z1b, d0r, d1r, y2r, o):
        r0 = 1.0 / jnp.maximum(d0r[:, 0:1], 1.0)
        r1 = 1.0 / jnp.maximum(d1r[:, 0:1], 1.0)
        z0 = jnp.concatenate([z0a[...], z0b[...]], axis=1)
        z1 = jnp.concatenate([z1a[...], z1b[...]], axis=1)
        o[...] = y2r[...] + z0 * r0 + z1 * r1

    in_specs = (
        [pl.BlockSpec((MT, W64), lambda i: (i, 0))] * 4
        + [pl.BlockSpec((MT, 16), lambda i: (i, 0))] * 2
        + [pl.BlockSpec((MT, 128), lambda i: (i, 0))]
    )
    return pl.pallas_call(
        body, grid=(GRID_M,), in_specs=in_specs,
        out_specs=pl.BlockSpec((MT, 128), lambda i: (i, 0)),
        out_shape=jax.ShapeDtypeStruct((N, 128), f32),
    )(*zpieces, deg0, deg1, y2)


def kernel(x, rel0_src, rel0_dst, rel1_src, rel1_dst,
           W0_r0, W0_r1, Wl0, b0, W1_r0, W1_r1, Wl1, b1,
           W2_r0, W2_r1, Wl2, b2):
    s0, d0 = _prep(rel0_src, rel0_dst)
    s1, d1 = _prep(rel1_src, rel1_dst)
    srcs = jnp.stack([s0, s1])
    dsts = jnp.stack([d0, d1])
    zrows = jnp.zeros((ROWS_PER, W64), f32)
    zrows16 = jnp.zeros((ROWS_PER, 16), f32)
    ones16 = jnp.ones((CHUNK, 16), f32)

    deg0, deg1 = _deg2(dsts, ones16, zrows16)

    xa, xb = x[:, :W64], x[:, W64:]
    a0a, a1a = _seg2(xa, xa, srcs, dsts, zrows)
    a0b, a1b = _seg2(xb, xb, srcs, dsts, zrows)
    h1 = _dense_combine(
        [a0a, a0b, a1a, a1b, x], [0, 0, 1, 1, None], [deg0, deg1],
        [W0_r0[:W64], W0_r0[W64:], W0_r1[:W64], W0_r1[W64:], Wl0],
        b0, True, (W64,) * 4)
    h10, h11, h12, h13 = h1

    b0a, b1a = _seg2(h10, h10, srcs, dsts, zrows)
    b0b, b1b = _seg2(h11, h11, srcs, dsts, zrows)
    b0c, b1c = _seg2(h12, h12, srcs, dsts, zrows)
    b0d, b1d = _seg2(h13, h13, srcs, dsts, zrows)
    h2 = _dense_combine(
        [b0a, b0b, b0c, b0d, b1a, b1b, b1c, b1d, h10, h11, h12, h13],
        [0, 0, 0, 0, 1, 1, 1, 1, None, None, None, None],
        [deg0, deg1],
        [W1_r0[:W64], W1_r0[W64:128], W1_r0[128:192], W1_r0[192:],
         W1_r1[:W64], W1_r1[W64:128], W1_r1[128:192], W1_r1[192:],
         Wl1[:W64], Wl1[W64:128], Wl1[128:192], Wl1[192:]],
        b1, True, (W64,) * 4)
    h20, h21, h22, h23 = h2

    hmats = [h20, h21, h22, h23]
    wrows = [(0, W64), (W64, 128), (128, 192), (192, 256)]
    y0a, y0b, y1a, y1b, y2 = _premm(
        hmats,
        [W2_r0[a:bb] for a, bb in wrows],
        [W2_r1[a:bb] for a, bb in wrows],
        [Wl2[a:bb] for a, bb in wrows],
        b2)
    z0a, z1a = _seg2(y0a, y1a, srcs, dsts, zrows)
    z0b, z1b = _seg2(y0b, y1b, srcs, dsts, zrows)
    return _final([z0a, z0b, z1a, z1b], deg0, deg1, y2)


# speedup vs baseline: 2.1760x; 1.0929x over previous
import functools

import jax
import jax.numpy as jnp
from jax import lax
from jax.experimental import pallas as pl
from jax.experimental.pallas import tpu as pltpu
from jax.experimental.pallas import tpu_sc as plsc

N = 10000
E = 160000
NSUB = 16
CHUNK = 128
NCHUNK = 80
EPT = NCHUNK * CHUNK
PADE = NSUB * EPT
ROWS_PER = 632
NROWS = NSUB * ROWS_PER
W64 = 64
NSLOT = 4
LOOKA = 2
TROWS = N // NSUB
PBITS = 14
PMASK = (1 << PBITS) - 1
MT = 400
GRID_M = N // MT

f32 = jnp.float32
_mesh = plsc.VectorSubcoreMesh(core_axis_name="c", subcore_axis_name="s")


def _make_seg(np_):
    @functools.partial(
        pl.kernel,
        out_type=(jax.ShapeDtypeStruct((NROWS, W64), f32),) * (2 * np_),
        mesh=_mesh,
        scratch_types=[
            pltpu.VMEM((NCHUNK, CHUNK), jnp.int32),
            pltpu.VMEM((NSLOT, CHUNK), jnp.int32),
            pltpu.VMEM((NSLOT, CHUNK), jnp.int32),
            pltpu.VMEM((NSLOT, CHUNK, W64), f32),
            pltpu.VMEM_SHARED((NROWS, W64), f32),
            pltpu.VMEM_SHARED((N, W64), f32),
            pltpu.SemaphoreType.DMA((NSLOT,)),
            pltpu.SemaphoreType.DMA((NSLOT,)),
        ],
        compiler_params=pltpu.CompilerParams(use_tc_tiling_on_sc=False),
    )
    def seg(*refs):
        tabs = refs[:2 * np_]
        packed, zrows = refs[2 * np_:2 * np_ + 2]
        outs = refs[2 * np_ + 2:4 * np_ + 2]
        pslab, sidxr, didxr, bufs, acc, tsp, gsems, ssems = refs[4 * np_ + 2:]
        c = lax.axis_index("c")
        s = lax.axis_index("s")
        row0 = s * ROWS_PER
        pltpu.sync_copy(packed.at[c, s], pslab)

        def unpack(j, b):
            for q in range(CHUNK // 16):
                v = pslab[j, pl.ds(q * 16, 16)]
                sidxr[b, pl.ds(q * 16, 16)] = lax.shift_right_logical(v, PBITS)
                didxr[b, pl.ds(q * 16, 16)] = lax.bitwise_and(v, PMASK)

        def gath(b):
            return pltpu.make_async_copy(tsp.at[sidxr.at[b]], bufs.at[b],
                                         gsems.at[b])

        def scat(b):
            return pltpu.make_async_copy(bufs.at[b], acc.at[didxr.at[b]],
                                         ssems.at[b])

        def body_pass(p, carry):
            pltpu.sync_copy(zrows, acc.at[pl.ds(row0, ROWS_PER)])
            for pi in range(np_):

                @pl.when(p == pi)
                def _():
                    @pl.when(c == 0)
                    def _():
                        pltpu.sync_copy(tabs[2 * pi].at[pl.ds(s * TROWS, TROWS)],
                                        tsp.at[pl.ds(s * TROWS, TROWS)])

                    @pl.when(c == 1)
                    def _():
                        pltpu.sync_copy(
                            tabs[2 * pi + 1].at[pl.ds(s * TROWS, TROWS)],
                            tsp.at[pl.ds(s * TROWS, TROWS)])

            plsc.subcore_barrier()
            for b in range(LOOKA):
                unpack(b, b)
                gath(b).start()

            def body(gg, carry2):
                j0 = gg * NSLOT
                for b in range(NSLOT):
                    j = j0 + b
                    gath(b).wait()
                    pltpu.async_copy(bufs.at[b], acc.at[didxr.at[b]],
                                     ssems.at[b], add=True)

                    @pl.when(j + LOOKA < NCHUNK)
                    def _():
                        bl = (b + LOOKA) % NSLOT

                        @pl.when(j >= NSLOT - LOOKA)
                        def _():
                            scat(bl).wait()

                        unpack(j + LOOKA, bl)
                        gath(bl).start()
                return carry2

            lax.fori_loop(0, NCHUNK // NSLOT, body, 0)
            for b in range(NSLOT):
                scat(b).wait()
            plsc.subcore_barrier()
            for pi in range(np_):

                @pl.when(p == pi)
                def _():
                    @pl.when(c == 0)
                    def _():
                        pltpu.sync_copy(acc.at[pl.ds(row0, ROWS_PER)],
                                        outs[2 * pi].at[pl.ds(row0, ROWS_PER)])

                    @pl.when(c == 1)
                    def _():
                        pltpu.sync_copy(
                            acc.at[pl.ds(row0, ROWS_PER)],
                            outs[2 * pi + 1].at[pl.ds(row0, ROWS_PER)])

            return carry

        lax.fori_loop(0, np_, body_pass, 0)

    return seg


_seg_x2 = _make_seg(2)
_seg_x4 = _make_seg(4)


@functools.partial(
    pl.kernel,
    out_type=(jax.ShapeDtypeStruct((NROWS, 16), f32),
              jax.ShapeDtypeStruct((NROWS, 16), f32)),
    mesh=_mesh,
    scratch_types=[
        pltpu.VMEM((NCHUNK, CHUNK), jnp.int32),
        pltpu.VMEM((CHUNK,), jnp.int32),
        pltpu.VMEM((CHUNK, 16), f32),
        pltpu.VMEM_SHARED((NROWS, 16), f32),
    ],
    compiler_params=pltpu.CompilerParams(use_tc_tiling_on_sc=False),
)
def _deg2(packed, ones_rows, zrows, out0, out1, pslab, didxv, onesv, acc):
    c = lax.axis_index("c")
    s = lax.axis_index("s")
    row0 = s * ROWS_PER
    pltpu.sync_copy(packed.at[c, s], pslab)
    pltpu.sync_copy(ones_rows, onesv)
    pltpu.sync_copy(zrows, acc.at[pl.ds(row0, ROWS_PER)])
    plsc.subcore_barrier()

    def body(j, carry):
        for q in range(CHUNK // 16):
            v = pslab[j, pl.ds(q * 16, 16)]
            didxv[pl.ds(q * 16, 16)] = lax.bitwise_and(v, PMASK)
        pltpu.sync_copy(onesv, acc.at[didxv], add=True)
        return carry

    lax.fori_loop(0, NCHUNK, body, 0)
    plsc.subcore_barrier()

    @pl.when(c == 0)
    def _():
        pltpu.sync_copy(acc.at[pl.ds(row0, ROWS_PER)],
                        out0.at[pl.ds(row0, ROWS_PER)])

    @pl.when(c == 1)
    def _():
        pltpu.sync_copy(acc.at[pl.ds(row0, ROWS_PER)],
                        out1.at[pl.ds(row0, ROWS_PER)])


def _prep(src, dst):
    pad = PADE - E
    word = src.astype(jnp.int32) * (1 << PBITS) + dst.astype(jnp.int32)
    word = jnp.concatenate([word, jnp.full((pad,), N, jnp.int32)])
    return word.reshape(NSUB, NCHUNK, CHUNK)


def _dense_combine(mats, scaled_by, degs, Ws, b, act, out_widths):
    OUT = Ws[0].shape[1]
    nm = len(mats)
    nd = len(degs)

    def body(*refs):
        mrefs = refs[:nm]
        drefs = refs[nm:nm + nd]
        wrefs = refs[nm + nd:nm + nd + nm]
        bref = refs[nm + nd + nm]
        orefs = refs[nm + nd + nm + 1:]
        rs = [1.0 / jnp.maximum(dr[:, 0:1], 1.0) for dr in
              [d[...] for d in drefs]]
        res = jnp.zeros((MT, OUT), f32)
        for mref, sb, wref in zip(mrefs, scaled_by, wrefs):
            xm = mref[...]
            if sb is not None:
                xm = xm * rs[sb]
            res = res + jnp.dot(xm, wref[...], preferred_element_type=f32)
        res = res + bref[...]
        if act:
            res = jnp.maximum(res, 0.0)
        off = 0
        for oref, w in zip(orefs, out_widths):
            oref[...] = res[:, off:off + w]
            off += w

    in_specs = (
        [pl.BlockSpec((MT, m.shape[1]), lambda i: (i, 0)) for m in mats]
        + [pl.BlockSpec((MT, 16), lambda i: (i, 0)) for _ in degs]
        + [pl.BlockSpec(w.shape, lambda i: (0, 0)) for w in Ws]
        + [pl.BlockSpec((1, OUT), lambda i: (0, 0))]
    )
    out_shape = tuple(jax.ShapeDtypeStruct((N, w), f32) for w in out_widths)
    out_specs = tuple(pl.BlockSpec((MT, w), lambda i: (i, 0))
                      for w in out_widths)
    res = pl.pallas_call(
        body, grid=(GRID_M,), in_specs=in_specs, out_specs=out_specs,
        out_shape=out_shape,
    )(*mats, *degs, *Ws, b.reshape(1, OUT))
    return res


def _premm(hmats, W0s, W1s, Wls, b):
    nh = len(hmats)

    def body(*refs):
        hrefs = refs[:nh]
        w0refs = refs[nh:2 * nh]
        w1refs = refs[2 * nh:3 * nh]
        wlrefs = refs[3 * nh:4 * nh]
        bref = refs[4 * nh]
        o0a, o0b, o1a, o1b, o2 = refs[4 * nh + 1:]
        hs = [h[...] for h in hrefs]
        r0 = sum(jnp.dot(h, w[...], preferred_element_type=f32)
                 for h, w in zip(hs, w0refs))
        r1 = sum(jnp.dot(h, w[...], preferred_element_type=f32)
                 for h, w in zip(hs, w1refs))
        r2 = sum(jnp.dot(h, w[...], preferred_element_type=f32)
                 for h, w in zip(hs, wlrefs)) + bref[...]
        o0a[...] = r0[:, :W64]
        o0b[...] = r0[:, W64:]
        o1a[...] = r1[:, :W64]
        o1b[...] = r1[:, W64:]
        o2[...] = r2

    in_specs = (
        [pl.BlockSpec((MT, W64), lambda i: (i, 0)) for _ in hmats]
        + [pl.BlockSpec(w.shape, lambda i: (0, 0))
           for w in (*W0s, *W1s, *Wls)]
        + [pl.BlockSpec((1, 128), lambda i: (0, 0))]
    )
    out_shape = (tuple(jax.ShapeDtypeStruct((N, W64), f32) for _ in range(4))
                 + (jax.ShapeDtypeStruct((N, 128), f32),))
    out_specs = (tuple(pl.BlockSpec((MT, W64), lambda i: (i, 0))
                       for _ in range(4))
                 + (pl.BlockSpec((MT, 128), lambda i: (i, 0)),))
    return pl.pallas_call(
        body, grid=(GRID_M,), in_specs=in_specs, out_specs=out_specs,
        out_shape=out_shape,
    )(*hmats, *W0s, *W1s, *Wls, b.reshape(1, 128))


def _final(zpieces, deg0, deg1, y2):
    def body(z0a, z0b, z1a, z1b, d0r, d1r, y2r, o):
        r0 = 1.0 / jnp.maximum(d0r[:, 0:1], 1.0)
        r1 = 1.0 / jnp.maximum(d1r[:, 0:1], 1.0)
        z0 = jnp.concatenate([z0a[...], z0b[...]], axis=1)
        z1 = jnp.concatenate([z1a[...], z1b[...]], axis=1)
        o[...] = y2r[...] + z0 * r0 + z1 * r1

    in_specs = (
        [pl.BlockSpec((MT, W64), lambda i: (i, 0))] * 4
        + [pl.BlockSpec((MT, 16), lambda i: (i, 0))] * 2
        + [pl.BlockSpec((MT, 128), lambda i: (i, 0))]
    )
    return pl.pallas_call(
        body, grid=(GRID_M,), in_specs=in_specs,
        out_specs=pl.BlockSpec((MT, 128), lambda i: (i, 0)),
        out_shape=jax.ShapeDtypeStruct((N, 128), f32),
    )(*zpieces, deg0, deg1, y2)


def kernel(x, rel0_src, rel0_dst, rel1_src, rel1_dst,
           W0_r0, W0_r1, Wl0, b0, W1_r0, W1_r1, Wl1, b1,
           W2_r0, W2_r1, Wl2, b2):
    packed = jnp.stack([_prep(rel0_src, rel0_dst),
                        _prep(rel1_src, rel1_dst)])
    zrows = jnp.zeros((ROWS_PER, W64), f32)
    zrows16 = jnp.zeros((ROWS_PER, 16), f32)
    ones16 = jnp.ones((CHUNK, 16), f32)

    deg0, deg1 = _deg2(packed, ones16, zrows16)

    xa, xb = x[:, :W64], x[:, W64:]
    a0a, a1a, a0b, a1b = _seg_x2(xa, xa, xb, xb, packed, zrows)
    h1 = _dense_combine(
        [a0a, a0b, a1a, a1b, x], [0, 0, 1, 1, None], [deg0, deg1],
        [W0_r0[:W64], W0_r0[W64:], W0_r1[:W64], W0_r1[W64:], Wl0],
        b0, True, (W64,) * 4)
    h10, h11, h12, h13 = h1

    b0a, b1a, b0b, b1b, b0c, b1c, b0d, b1d = _seg_x4(
        h10, h10, h11, h11, h12, h12, h13, h13, packed, zrows)
    h2 = _dense_combine(
        [b0a, b0b, b0c, b0d, b1a, b1b, b1c, b1d, h10, h11, h12, h13],
        [0, 0, 0, 0, 1, 1, 1, 1, None, None, None, None],
        [deg0, deg1],
        [W1_r0[:W64], W1_r0[W64:128], W1_r0[128:192], W1_r0[192:],
         W1_r1[:W64], W1_r1[W64:128], W1_r1[128:192], W1_r1[192:],
         Wl1[:W64], Wl1[W64:128], Wl1[128:192], Wl1[192:]],
        b1, True, (W64,) * 4)
    h20, h21, h22, h23 = h2

    hmats = [h20, h21, h22, h23]
    wrows = [(0, W64), (W64, 128), (128, 192), (192, 256)]
    y0a, y0b, y1a, y1b, y2 = _premm(
        hmats,
        [W2_r0[a:bb] for a, bb in wrows],
        [W2_r1[a:bb] for a, bb in wrows],
        [Wl2[a:bb] for a, bb in wrows],
        b2)
    z0a, z1a, z0b, z1b = _seg_x2(y0a, y1a, y0b, y1b, packed, zrows)
    return _final([z0a, z0b, z1a, z1b], deg0, deg1, y2)


# speedup vs baseline: 2.3108x; 1.0620x over previous
import functools

import jax
import jax.numpy as jnp
from jax import lax
from jax.experimental import pallas as pl
from jax.experimental.pallas import tpu as pltpu
from jax.experimental.pallas import tpu_sc as plsc

N = 10000
E = 160000
NSUB = 16
CHUNK = 128
NCHUNK = 80
EPT = NCHUNK * CHUNK
PADE = NSUB * EPT
ROWS_PER = 632
NROWS = NSUB * ROWS_PER
W64 = 64
NSLOT = 4
LOOKA = 2
TROWS = N // NSUB
PBITS = 14
PMASK = (1 << PBITS) - 1
MT = 400
GRID_M = N // MT

f32 = jnp.float32
_mesh = plsc.VectorSubcoreMesh(core_axis_name="c", subcore_axis_name="s")


def _make_seg(np_):
    @functools.partial(
        pl.kernel,
        out_type=(jax.ShapeDtypeStruct((NROWS, W64), f32),) * (2 * np_),
        mesh=_mesh,
        scratch_types=[
            pltpu.VMEM((NCHUNK, CHUNK), jnp.int32),
            pltpu.VMEM((NSLOT, CHUNK), jnp.int32),
            pltpu.VMEM((NSLOT, CHUNK), jnp.int32),
            pltpu.VMEM((NSLOT, CHUNK, W64), f32),
            pltpu.VMEM_SHARED((NROWS, W64), f32),
            pltpu.VMEM_SHARED((N, W64), f32),
            pltpu.SemaphoreType.DMA((NSLOT,)),
            pltpu.SemaphoreType.DMA((NSLOT,)),
        ],
        compiler_params=pltpu.CompilerParams(use_tc_tiling_on_sc=False),
    )
    def seg(*refs):
        tabs = refs[:2 * np_]
        packed, zrows = refs[2 * np_:2 * np_ + 2]
        outs = refs[2 * np_ + 2:4 * np_ + 2]
        pslab, sidxr, didxr, bufs, acc, tsp, gsems, ssems = refs[4 * np_ + 2:]
        c = lax.axis_index("c")
        s = lax.axis_index("s")
        row0 = s * ROWS_PER
        pltpu.sync_copy(packed.at[c, s], pslab)

        def unpack(j, b):
            for q in range(CHUNK // 16):
                v = pslab[j, pl.ds(q * 16, 16)]
                sidxr[b, pl.ds(q * 16, 16)] = lax.shift_right_logical(v, PBITS)
                didxr[b, pl.ds(q * 16, 16)] = lax.bitwise_and(v, PMASK)

        def gath(b):
            return pltpu.make_async_copy(tsp.at[sidxr.at[b]], bufs.at[b],
                                         gsems.at[b])

        def scat(b):
            return pltpu.make_async_copy(bufs.at[b], acc.at[didxr.at[b]],
                                         ssems.at[b])

        def body_pass(p, carry):
            pltpu.sync_copy(zrows, acc.at[pl.ds(row0, ROWS_PER)])
            for pi in range(np_):

                @pl.when(p == pi)
                def _():
                    @pl.when(c == 0)
                    def _():
                        pltpu.sync_copy(tabs[2 * pi].at[pl.ds(s * TROWS, TROWS)],
                                        tsp.at[pl.ds(s * TROWS, TROWS)])

                    @pl.when(c == 1)
                    def _():
                        pltpu.sync_copy(
                            tabs[2 * pi + 1].at[pl.ds(s * TROWS, TROWS)],
                            tsp.at[pl.ds(s * TROWS, TROWS)])

            plsc.subcore_barrier()
            for b in range(LOOKA):
                unpack(b, b)
                gath(b).start()

            def body(gg, carry2):
                j0 = gg * NSLOT
                for b in range(NSLOT):
                    j = j0 + b
                    gath(b).wait()
                    pltpu.async_copy(bufs.at[b], acc.at[didxr.at[b]],
                                     ssems.at[b], add=True)

                    @pl.when(j + LOOKA < NCHUNK)
                    def _():
                        bl = (b + LOOKA) % NSLOT

                        @pl.when(j >= NSLOT - LOOKA)
                        def _():
                            scat(bl).wait()

                        unpack(j + LOOKA, bl)
                        gath(bl).start()
                return carry2

            lax.fori_loop(0, NCHUNK // NSLOT, body, 0)
            for b in range(NSLOT):
                scat(b).wait()
            plsc.subcore_barrier()
            for pi in range(np_):

                @pl.when(p == pi)
                def _():
                    @pl.when(c == 0)
                    def _():
                        pltpu.sync_copy(acc.at[pl.ds(row0, ROWS_PER)],
                                        outs[2 * pi].at[pl.ds(row0, ROWS_PER)])

                    @pl.when(c == 1)
                    def _():
                        pltpu.sync_copy(
                            acc.at[pl.ds(row0, ROWS_PER)],
                            outs[2 * pi + 1].at[pl.ds(row0, ROWS_PER)])

            return carry

        lax.fori_loop(0, np_, body_pass, 0)

    return seg


_seg_x2 = _make_seg(2)
_seg_x4 = _make_seg(4)


@functools.partial(
    pl.kernel,
    out_type=(jax.ShapeDtypeStruct((NROWS, 16), f32),
              jax.ShapeDtypeStruct((NROWS, 16), f32)),
    mesh=_mesh,
    scratch_types=[
        pltpu.VMEM((NCHUNK, CHUNK), jnp.int32),
        pltpu.VMEM((CHUNK,), jnp.int32),
        pltpu.VMEM((CHUNK, 16), f32),
        pltpu.VMEM_SHARED((NROWS, 16), f32),
    ],
    compiler_params=pltpu.CompilerParams(use_tc_tiling_on_sc=False),
)
def _deg2(packed, ones_rows, zrows, out0, out1, pslab, didxv, onesv, acc):
    c = lax.axis_index("c")
    s = lax.axis_index("s")
    row0 = s * ROWS_PER
    pltpu.sync_copy(packed.at[c, s], pslab)
    pltpu.sync_copy(ones_rows, onesv)
    pltpu.sync_copy(zrows, acc.at[pl.ds(row0, ROWS_PER)])
    plsc.subcore_barrier()

    def body(j, carry):
        for q in range(CHUNK // 16):
            v = pslab[j, pl.ds(q * 16, 16)]
            didxv[pl.ds(q * 16, 16)] = lax.bitwise_and(v, PMASK)
        pltpu.sync_copy(onesv, acc.at[didxv], add=True)
        return carry

    lax.fori_loop(0, NCHUNK, body, 0)
    plsc.subcore_barrier()

    @pl.when(c == 0)
    def _():
        pltpu.sync_copy(acc.at[pl.ds(row0, ROWS_PER)],
                        out0.at[pl.ds(row0, ROWS_PER)])

    @pl.when(c == 1)
    def _():
        pltpu.sync_copy(acc.at[pl.ds(row0, ROWS_PER)],
                        out1.at[pl.ds(row0, ROWS_PER)])


def _prep(src, dst):
    pad = PADE - E
    word = src.astype(jnp.int32) * (1 << PBITS) + dst.astype(jnp.int32)
    word = jnp.concatenate([word, jnp.full((pad,), N, jnp.int32)])
    return word.reshape(NSUB, NCHUNK, CHUNK)


def _dense_combine(mats, scaled_by, degs, Ws, b, act, out_widths, post=None):
    OUT = Ws[0].shape[1]
    nm = len(mats)
    nd = len(degs)
    npost = 4 if post is not None else 0

    def body(*refs):
        mrefs = refs[:nm]
        drefs = refs[nm:nm + nd]
        wrefs = refs[nm + nd:nm + nd + nm]
        bref = refs[nm + nd + nm]
        prefs = refs[nm + nd + nm + 1:nm + nd + nm + 1 + npost]
        orefs = refs[nm + nd + nm + 1 + npost:]
        rs = [1.0 / jnp.maximum(dr[:, 0:1], 1.0) for dr in
              [d[...] for d in drefs]]
        res = jnp.zeros((MT, OUT), f32)
        for mref, sb, wref in zip(mrefs, scaled_by, wrefs):
            xm = mref[...]
            if sb is not None:
                xm = xm * rs[sb]
            res = res + jnp.dot(xm, wref[...], preferred_element_type=f32)
        res = res + bref[...]
        if act:
            res = jnp.maximum(res, 0.0)
        if post is not None:
            y0 = jnp.dot(res, prefs[0][...], preferred_element_type=f32)
            y1 = jnp.dot(res, prefs[1][...], preferred_element_type=f32)
            y2 = (jnp.dot(res, prefs[2][...], preferred_element_type=f32)
                  + prefs[3][...])
            orefs[0][...] = y0[:, :W64]
            orefs[1][...] = y0[:, W64:]
            orefs[2][...] = y1[:, :W64]
            orefs[3][...] = y1[:, W64:]
            orefs[4][...] = y2
        else:
            off = 0
            for oref, w in zip(orefs, out_widths):
                oref[...] = res[:, off:off + w]
                off += w

    pargs = []
    pspecs = []
    if post is not None:
        P0, P1, Pl, pb = post
        pargs = [P0, P1, Pl, pb.reshape(1, 128)]
        pspecs = [pl.BlockSpec(P0.shape, lambda i: (0, 0)),
                  pl.BlockSpec(P1.shape, lambda i: (0, 0)),
                  pl.BlockSpec(Pl.shape, lambda i: (0, 0)),
                  pl.BlockSpec((1, 128), lambda i: (0, 0))]

    in_specs = (
        [pl.BlockSpec((MT, m.shape[1]), lambda i: (i, 0)) for m in mats]
        + [pl.BlockSpec((MT, 16), lambda i: (i, 0)) for _ in degs]
        + [pl.BlockSpec(w.shape, lambda i: (0, 0)) for w in Ws]
        + [pl.BlockSpec((1, OUT), lambda i: (0, 0))]
        + pspecs
    )
    out_shape = tuple(jax.ShapeDtypeStruct((N, w), f32) for w in out_widths)
    out_specs = tuple(pl.BlockSpec((MT, w), lambda i: (i, 0))
                      for w in out_widths)
    res = pl.pallas_call(
        body, grid=(GRID_M,), in_specs=in_specs, out_specs=out_specs,
        out_shape=out_shape,
    )(*mats, *degs, *Ws, b.reshape(1, OUT), *pargs)
    return res


def _final(zpieces, deg0, deg1, y2):
    def body(z0a, z0b, z1a, z1b, d0r, d1r, y2r, o):
        r0 = 1.0 / jnp.maximum(d0r[:, 0:1], 1.0)
        r1 = 1.0 / jnp.maximum(d1r[:, 0:1], 1.0)
        z0 = jnp.concatenate([z0a[...], z0b[...]], axis=1)
        z1 = jnp.concatenate([z1a[...], z1b[...]], axis=1)
        o[...] = y2r[...] + z0 * r0 + z1 * r1

    in_specs = (
        [pl.BlockSpec((MT, W64), lambda i: (i, 0))] * 4
        + [pl.BlockSpec((MT, 16), lambda i: (i, 0))] * 2
        + [pl.BlockSpec((MT, 128), lambda i: (i, 0))]
    )
    return pl.pallas_call(
        body, grid=(GRID_M,), in_specs=in_specs,
        out_specs=pl.BlockSpec((MT, 128), lambda i: (i, 0)),
        out_shape=jax.ShapeDtypeStruct((N, 128), f32),
    )(*zpieces, deg0, deg1, y2)


def kernel(x, rel0_src, rel0_dst, rel1_src, rel1_dst,
           W0_r0, W0_r1, Wl0, b0, W1_r0, W1_r1, Wl1, b1,
           W2_r0, W2_r1, Wl2, b2):
    packed = jnp.stack([_prep(rel0_src, rel0_dst),
                        _prep(rel1_src, rel1_dst)])
    zrows = jnp.zeros((ROWS_PER, W64), f32)
    zrows16 = jnp.zeros((ROWS_PER, 16), f32)
    ones16 = jnp.ones((CHUNK, 16), f32)

    deg0, deg1 = _deg2(packed, ones16, zrows16)

    xa, xb = x[:, :W64], x[:, W64:]
    a0a, a1a, a0b, a1b = _seg_x2(xa, xa, xb, xb, packed, zrows)
    h1 = _dense_combine(
        [a0a, a0b, a1a, a1b, x], [0, 0, 1, 1, None], [deg0, deg1],
        [W0_r0[:W64], W0_r0[W64:], W0_r1[:W64], W0_r1[W64:], Wl0],
        b0, True, (W64,) * 4)
    h10, h11, h12, h13 = h1

    b0a, b1a, b0b, b1b, b0c, b1c, b0d, b1d = _seg_x4(
        h10, h10, h11, h11, h12, h12, h13, h13, packed, zrows)
    y0a, y0b, y1a, y1b, y2 = _dense_combine(
        [b0a, b0b, b0c, b0d, b1a, b1b, b1c, b1d, h10, h11, h12, h13],
        [0, 0, 0, 0, 1, 1, 1, 1, None, None, None, None],
        [deg0, deg1],
        [W1_r0[:W64], W1_r0[W64:128], W1_r0[128:192], W1_r0[192:],
         W1_r1[:W64], W1_r1[W64:128], W1_r1[128:192], W1_r1[192:],
         Wl1[:W64], Wl1[W64:128], Wl1[128:192], Wl1[192:]],
        b1, True, (W64, W64, W64, W64, 128),
        post=(W2_r0, W2_r1, Wl2, b2))
    z0a, z1a, z0b, z1b = _seg_x2(y0a, y1a, y0b, y1b, packed, zrows)
    return _final([z0a, z0b, z1a, z1b], deg0, deg1, y2)


# speedup vs baseline: 2.3488x; 1.0164x over previous
import functools

import jax
import jax.numpy as jnp
from jax import lax
from jax.experimental import pallas as pl
from jax.experimental.pallas import tpu as pltpu
from jax.experimental.pallas import tpu_sc as plsc

N = 10000
E = 160000
NSUB = 16
CHUNK = 128
NCHUNK = 80
EPT = NCHUNK * CHUNK
PADE = NSUB * EPT
ROWS_PER = 632
NROWS = NSUB * ROWS_PER
W64 = 64
NSLOT = 4
LOOKA = 2
TROWS = N // NSUB
PBITS = 14
PMASK = (1 << PBITS) - 1
MT = 400
GRID_M = N // MT

f32 = jnp.float32
_mesh = plsc.VectorSubcoreMesh(core_axis_name="c", subcore_axis_name="s")


def _make_seg(np_):
    @functools.partial(
        pl.kernel,
        out_type=(jax.ShapeDtypeStruct((NROWS, W64), f32),) * (2 * np_),
        mesh=_mesh,
        scratch_types=[
            pltpu.VMEM((NCHUNK, CHUNK), jnp.int32),
            pltpu.VMEM((NSLOT, CHUNK), jnp.int32),
            pltpu.VMEM((NSLOT, CHUNK), jnp.int32),
            pltpu.VMEM((NSLOT, CHUNK, W64), f32),
            pltpu.VMEM((ROWS_PER // 8, W64), f32),
            pltpu.VMEM_SHARED((NROWS, W64), f32),
            pltpu.VMEM_SHARED((N, W64), f32),
            pltpu.SemaphoreType.DMA((NSLOT,)),
            pltpu.SemaphoreType.DMA((NSLOT,)),
        ],
        compiler_params=pltpu.CompilerParams(use_tc_tiling_on_sc=False),
    )
    def seg(*refs):
        tabs = refs[:2 * np_]
        packed, zrows = refs[2 * np_:2 * np_ + 2]
        outs = refs[2 * np_ + 2:4 * np_ + 2]
        (pslab, sidxr, didxr, bufs, zbuf, acc, tsp,
         gsems, ssems) = refs[4 * np_ + 2:]
        c = lax.axis_index("c")
        s = lax.axis_index("s")
        row0 = s * ROWS_PER
        pltpu.sync_copy(packed.at[c, s], pslab)
        pltpu.sync_copy(zrows, zbuf)
        zstep = ROWS_PER // 8

        def unpack(j, b):
            for q in range(CHUNK // 16):
                v = pslab[j, pl.ds(q * 16, 16)]
                sidxr[b, pl.ds(q * 16, 16)] = lax.shift_right_logical(v, PBITS)
                didxr[b, pl.ds(q * 16, 16)] = lax.bitwise_and(v, PMASK)

        def gath(b):
            return pltpu.make_async_copy(tsp.at[sidxr.at[b]], bufs.at[b],
                                         gsems.at[b])

        def scat(b):
            return pltpu.make_async_copy(bufs.at[b], acc.at[didxr.at[b]],
                                         ssems.at[b])

        def body_pass(p, carry):
            for k in range(8):
                sem = gsems if k < 4 else ssems
                pltpu.async_copy(zbuf, acc.at[pl.ds(row0 + k * zstep, zstep)],
                                 sem.at[k % NSLOT])
            for k in range(8):
                sem = gsems if k < 4 else ssems
                pltpu.make_async_copy(
                    zbuf, acc.at[pl.ds(row0 + k * zstep, zstep)],
                    sem.at[k % NSLOT]).wait()
            for pi in range(np_):

                @pl.when(p == pi)
                def _():
                    @pl.when(c == 0)
                    def _():
                        pltpu.sync_copy(tabs[2 * pi].at[pl.ds(s * TROWS, TROWS)],
                                        tsp.at[pl.ds(s * TROWS, TROWS)])

                    @pl.when(c == 1)
                    def _():
                        pltpu.sync_copy(
                            tabs[2 * pi + 1].at[pl.ds(s * TROWS, TROWS)],
                            tsp.at[pl.ds(s * TROWS, TROWS)])

            plsc.subcore_barrier()
            for b in range(LOOKA):
                unpack(b, b)
                gath(b).start()

            def body(gg, carry2):
                j0 = gg * NSLOT
                for b in range(NSLOT):
                    j = j0 + b
                    gath(b).wait()
                    pltpu.async_copy(bufs.at[b], acc.at[didxr.at[b]],
                                     ssems.at[b], add=True)

                    @pl.when(j + LOOKA < NCHUNK)
                    def _():
                        bl = (b + LOOKA) % NSLOT

                        @pl.when(j >= NSLOT - LOOKA)
                        def _():
                            scat(bl).wait()

                        unpack(j + LOOKA, bl)
                        gath(bl).start()
                return carry2

            lax.fori_loop(0, NCHUNK // NSLOT, body, 0)
            for b in range(NSLOT):
                scat(b).wait()
            plsc.subcore_barrier()
            for pi in range(np_):

                @pl.when(p == pi)
                def _():
                    @pl.when(c == 0)
                    def _():
                        pltpu.sync_copy(acc.at[pl.ds(row0, ROWS_PER)],
                                        outs[2 * pi].at[pl.ds(row0, ROWS_PER)])

                    @pl.when(c == 1)
                    def _():
                        pltpu.sync_copy(
                            acc.at[pl.ds(row0, ROWS_PER)],
                            outs[2 * pi + 1].at[pl.ds(row0, ROWS_PER)])

            return carry

        lax.fori_loop(0, np_, body_pass, 0)

    return seg


_seg_x2 = _make_seg(2)
_seg_x4 = _make_seg(4)


@functools.partial(
    pl.kernel,
    out_type=(jax.ShapeDtypeStruct((NROWS, 16), f32),
              jax.ShapeDtypeStruct((NROWS, 16), f32)),
    mesh=_mesh,
    scratch_types=[
        pltpu.VMEM((NCHUNK, CHUNK), jnp.int32),
        pltpu.VMEM((CHUNK,), jnp.int32),
        pltpu.VMEM((CHUNK, 16), f32),
        pltpu.VMEM_SHARED((NROWS, 16), f32),
    ],
    compiler_params=pltpu.CompilerParams(use_tc_tiling_on_sc=False),
)
def _deg2(packed, ones_rows, zrows, out0, out1, pslab, didxv, onesv, acc):
    c = lax.axis_index("c")
    s = lax.axis_index("s")
    row0 = s * ROWS_PER
    pltpu.sync_copy(packed.at[c, s], pslab)
    pltpu.sync_copy(ones_rows, onesv)
    pltpu.sync_copy(zrows, acc.at[pl.ds(row0, ROWS_PER)])
    plsc.subcore_barrier()

    def body(j, carry):
        for q in range(CHUNK // 16):
            v = pslab[j, pl.ds(q * 16, 16)]
            didxv[pl.ds(q * 16, 16)] = lax.bitwise_and(v, PMASK)
        pltpu.sync_copy(onesv, acc.at[didxv], add=True)
        return carry

    lax.fori_loop(0, NCHUNK, body, 0)
    plsc.subcore_barrier()

    @pl.when(c == 0)
    def _():
        pltpu.sync_copy(acc.at[pl.ds(row0, ROWS_PER)],
                        out0.at[pl.ds(row0, ROWS_PER)])

    @pl.when(c == 1)
    def _():
        pltpu.sync_copy(acc.at[pl.ds(row0, ROWS_PER)],
                        out1.at[pl.ds(row0, ROWS_PER)])


def _prep(src, dst):
    pad = PADE - E
    word = src.astype(jnp.int32) * (1 << PBITS) + dst.astype(jnp.int32)
    word = jnp.concatenate([word, jnp.full((pad,), N, jnp.int32)])
    return word.reshape(NSUB, NCHUNK, CHUNK)


def _dense_combine(mats, scaled_by, degs, Ws, b, act, out_widths, post=None):
    OUT = Ws[0].shape[1]
    nm = len(mats)
    nd = len(degs)
    npost = 4 if post is not None else 0

    def body(*refs):
        mrefs = refs[:nm]
        drefs = refs[nm:nm + nd]
        wrefs = refs[nm + nd:nm + nd + nm]
        bref = refs[nm + nd + nm]
        prefs = refs[nm + nd + nm + 1:nm + nd + nm + 1 + npost]
        orefs = refs[nm + nd + nm + 1 + npost:]
        rs = [1.0 / jnp.maximum(dr[:, 0:1], 1.0) for dr in
              [d[...] for d in drefs]]
        res = jnp.zeros((MT, OUT), f32)
        for mref, sb, wref in zip(mrefs, scaled_by, wrefs):
            xm = mref[...]
            if sb is not None:
                xm = xm * rs[sb]
            res = res + jnp.dot(xm, wref[...], preferred_element_type=f32)
        res = res + bref[...]
        if act:
            res = jnp.maximum(res, 0.0)
        if post is not None:
            y0 = jnp.dot(res, prefs[0][...], preferred_element_type=f32)
            y1 = jnp.dot(res, prefs[1][...], preferred_element_type=f32)
            y2 = (jnp.dot(res, prefs[2][...], preferred_element_type=f32)
                  + prefs[3][...])
            orefs[0][...] = y0[:, :W64]
            orefs[1][...] = y0[:, W64:]
            orefs[2][...] = y1[:, :W64]
            orefs[3][...] = y1[:, W64:]
            orefs[4][...] = y2
        else:
            off = 0
            for oref, w in zip(orefs, out_widths):
                oref[...] = res[:, off:off + w]
                off += w

    pargs = []
    pspecs = []
    if post is not None:
        P0, P1, Pl, pb = post
        pargs = [P0, P1, Pl, pb.reshape(1, 128)]
        pspecs = [pl.BlockSpec(P0.shape, lambda i: (0, 0)),
                  pl.BlockSpec(P1.shape, lambda i: (0, 0)),
                  pl.BlockSpec(Pl.shape, lambda i: (0, 0)),
                  pl.BlockSpec((1, 128), lambda i: (0, 0))]

    in_specs = (
        [pl.BlockSpec((MT, m.shape[1]), lambda i: (i, 0)) for m in mats]
        + [pl.BlockSpec((MT, 16), lambda i: (i, 0)) for _ in degs]
        + [pl.BlockSpec(w.shape, lambda i: (0, 0)) for w in Ws]
        + [pl.BlockSpec((1, OUT), lambda i: (0, 0))]
        + pspecs
    )
    out_shape = tuple(jax.ShapeDtypeStruct((N, w), f32) for w in out_widths)
    out_specs = tuple(pl.BlockSpec((MT, w), lambda i: (i, 0))
                      for w in out_widths)
    res = pl.pallas_call(
        body, grid=(GRID_M,), in_specs=in_specs, out_specs=out_specs,
        out_shape=out_shape,
    )(*mats, *degs, *Ws, b.reshape(1, OUT), *pargs)
    return res


def _final(zpieces, deg0, deg1, y2):
    def body(z0a, z0b, z1a, z1b, d0r, d1r, y2r, o):
        r0 = 1.0 / jnp.maximum(d0r[:, 0:1], 1.0)
        r1 = 1.0 / jnp.maximum(d1r[:, 0:1], 1.0)
        z0 = jnp.concatenate([z0a[...], z0b[...]], axis=1)
        z1 = jnp.concatenate([z1a[...], z1b[...]], axis=1)
        o[...] = y2r[...] + z0 * r0 + z1 * r1

    in_specs = (
        [pl.BlockSpec((MT, W64), lambda i: (i, 0))] * 4
        + [pl.BlockSpec((MT, 16), lambda i: (i, 0))] * 2
        + [pl.BlockSpec((MT, 128), lambda i: (i, 0))]
    )
    return pl.pallas_call(
        body, grid=(GRID_M,), in_specs=in_specs,
        out_specs=pl.BlockSpec((MT, 128), lambda i: (i, 0)),
        out_shape=jax.ShapeDtypeStruct((N, 128), f32),
    )(*zpieces, deg0, deg1, y2)


def kernel(x, rel0_src, rel0_dst, rel1_src, rel1_dst,
           W0_r0, W0_r1, Wl0, b0, W1_r0, W1_r1, Wl1, b1,
           W2_r0, W2_r1, Wl2, b2):
    packed = jnp.stack([_prep(rel0_src, rel0_dst),
                        _prep(rel1_src, rel1_dst)])
    zrows = jnp.zeros((ROWS_PER // 8, W64), f32)
    zrows16 = jnp.zeros((ROWS_PER, 16), f32)
    ones16 = jnp.ones((CHUNK, 16), f32)

    deg0, deg1 = _deg2(packed, ones16, zrows16)

    xa, xb = x[:, :W64], x[:, W64:]
    a0a, a1a, a0b, a1b = _seg_x2(xa, xa, xb, xb, packed, zrows)
    h1 = _dense_combine(
        [a0a, a0b, a1a, a1b, x], [0, 0, 1, 1, None], [deg0, deg1],
        [W0_r0[:W64], W0_r0[W64:], W0_r1[:W64], W0_r1[W64:], Wl0],
        b0, True, (W64,) * 4)
    h10, h11, h12, h13 = h1

    b0a, b1a, b0b, b1b, b0c, b1c, b0d, b1d = _seg_x4(
        h10, h10, h11, h11, h12, h12, h13, h13, packed, zrows)
    y0a, y0b, y1a, y1b, y2 = _dense_combine(
        [b0a, b0b, b0c, b0d, b1a, b1b, b1c, b1d, h10, h11, h12, h13],
        [0, 0, 0, 0, 1, 1, 1, 1, None, None, None, None],
        [deg0, deg1],
        [W1_r0[:W64], W1_r0[W64:128], W1_r0[128:192], W1_r0[192:],
         W1_r1[:W64], W1_r1[W64:128], W1_r1[128:192], W1_r1[192:],
         Wl1[:W64], Wl1[W64:128], Wl1[128:192], Wl1[192:]],
        b1, True, (W64, W64, W64, W64, 128),
        post=(W2_r0, W2_r1, Wl2, b2))
    z0a, z1a, z0b, z1b = _seg_x2(y0a, y1a, y0b, y1b, packed, zrows)
    return _final([z0a, z0b, z1a, z1b], deg0, deg1, y2)


# speedup vs baseline: 2.3915x; 1.0182x over previous
import functools

import jax
import jax.numpy as jnp
from jax import lax
from jax.experimental import pallas as pl
from jax.experimental.pallas import tpu as pltpu
from jax.experimental.pallas import tpu_sc as plsc

N = 10000
E = 160000
NSUB = 16
CHUNK = 128
NCHUNK = 80
EPT = NCHUNK * CHUNK
PADE = NSUB * EPT
ROWS_PER = 632
NROWS = NSUB * ROWS_PER
W64 = 64
NSLOT = 4
LOOKA = 2
TROWS = N // NSUB
PBITS = 14
PMASK = (1 << PBITS) - 1
MT = 400
GRID_M = N // MT

f32 = jnp.float32
_mesh = plsc.VectorSubcoreMesh(core_axis_name="c", subcore_axis_name="s")


def _make_seg(np_):
    @functools.partial(
        pl.kernel,
        out_type=(jax.ShapeDtypeStruct((NROWS, W64), f32),) * (2 * np_),
        mesh=_mesh,
        scratch_types=[
            pltpu.VMEM((NCHUNK, CHUNK), jnp.int32),
            pltpu.VMEM((NSLOT, CHUNK), jnp.int32),
            pltpu.VMEM((NSLOT, CHUNK), jnp.int32),
            pltpu.VMEM((NSLOT, CHUNK, W64), f32),
            pltpu.VMEM((ROWS_PER // 8, W64), f32),
            pltpu.VMEM_SHARED((NROWS, W64), f32),
            pltpu.VMEM_SHARED((N, W64), f32),
            pltpu.SemaphoreType.DMA((NSLOT,)),
            pltpu.SemaphoreType.DMA((NSLOT,)),
            pltpu.SemaphoreType.DMA,
            pltpu.SemaphoreType.DMA,
        ],
        compiler_params=pltpu.CompilerParams(use_tc_tiling_on_sc=False),
    )
    def seg(*refs):
        tabs = refs[:2 * np_]
        packed, zrows = refs[2 * np_:2 * np_ + 2]
        outs = refs[2 * np_ + 2:4 * np_ + 2]
        (pslab, sidxr, didxr, bufs, zbuf, acc, tsp,
         gsems, ssems, tsem, wsem) = refs[4 * np_ + 2:]
        c = lax.axis_index("c")
        s = lax.axis_index("s")
        row0 = s * ROWS_PER
        pltpu.sync_copy(packed.at[c, s], pslab)
        pltpu.sync_copy(zrows, zbuf)
        zstep = ROWS_PER // 8

        def stage_async(p):
            for pi in range(np_):

                @pl.when(p == pi)
                def _():
                    @pl.when(c == 0)
                    def _():
                        pltpu.async_copy(tabs[2 * pi].at[pl.ds(s * TROWS, TROWS)],
                                         tsp.at[pl.ds(s * TROWS, TROWS)], tsem)

                    @pl.when(c == 1)
                    def _():
                        pltpu.async_copy(
                            tabs[2 * pi + 1].at[pl.ds(s * TROWS, TROWS)],
                            tsp.at[pl.ds(s * TROWS, TROWS)], tsem)

        def wb_async(p):
            for pi in range(np_):

                @pl.when(p == pi)
                def _():
                    @pl.when(c == 0)
                    def _():
                        pltpu.async_copy(acc.at[pl.ds(row0, ROWS_PER)],
                                         outs[2 * pi].at[pl.ds(row0, ROWS_PER)],
                                         wsem)

                    @pl.when(c == 1)
                    def _():
                        pltpu.async_copy(
                            acc.at[pl.ds(row0, ROWS_PER)],
                            outs[2 * pi + 1].at[pl.ds(row0, ROWS_PER)], wsem)

        def wb_wait():
            pltpu.make_async_copy(acc.at[pl.ds(row0, ROWS_PER)],
                                  outs[0].at[pl.ds(row0, ROWS_PER)],
                                  wsem).wait()

        stage_async(0)

        def unpack(j, b):
            for q in range(CHUNK // 16):
                v = pslab[j, pl.ds(q * 16, 16)]
                sidxr[b, pl.ds(q * 16, 16)] = lax.shift_right_logical(v, PBITS)
                didxr[b, pl.ds(q * 16, 16)] = lax.bitwise_and(v, PMASK)

        def gath(b):
            return pltpu.make_async_copy(tsp.at[sidxr.at[b]], bufs.at[b],
                                         gsems.at[b])

        def scat(b):
            return pltpu.make_async_copy(bufs.at[b], acc.at[didxr.at[b]],
                                         ssems.at[b])

        def body_pass(p, carry):
            @pl.when(p > 0)
            def _():
                wb_wait()

            for k in range(8):
                sem = gsems if k < 4 else ssems
                pltpu.async_copy(zbuf, acc.at[pl.ds(row0 + k * zstep, zstep)],
                                 sem.at[k % NSLOT])
            pltpu.make_async_copy(tabs[0].at[pl.ds(s * TROWS, TROWS)],
                                  tsp.at[pl.ds(s * TROWS, TROWS)], tsem).wait()
            for k in range(8):
                sem = gsems if k < 4 else ssems
                pltpu.make_async_copy(
                    zbuf, acc.at[pl.ds(row0 + k * zstep, zstep)],
                    sem.at[k % NSLOT]).wait()
            plsc.subcore_barrier()
            for b in range(LOOKA):
                unpack(b, b)
                gath(b).start()

            def body(gg, carry2):
                j0 = gg * NSLOT
                for b in range(NSLOT):
                    j = j0 + b
                    gath(b).wait()
                    pltpu.async_copy(bufs.at[b], acc.at[didxr.at[b]],
                                     ssems.at[b], add=True)

                    @pl.when(j + LOOKA < NCHUNK)
                    def _():
                        bl = (b + LOOKA) % NSLOT

                        @pl.when(j >= NSLOT - LOOKA)
                        def _():
                            scat(bl).wait()

                        unpack(j + LOOKA, bl)
                        gath(bl).start()
                return carry2

            lax.fori_loop(0, NCHUNK // NSLOT, body, 0)
            for b in range(NSLOT):
                scat(b).wait()
            plsc.subcore_barrier()
            wb_async(p)

            @pl.when(p + 1 < np_)
            def _():
                stage_async(p + 1)

            return carry

        lax.fori_loop(0, np_, body_pass, 0)
        wb_wait()

    return seg


_seg_x2 = _make_seg(2)
_seg_x4 = _make_seg(4)


@functools.partial(
    pl.kernel,
    out_type=(jax.ShapeDtypeStruct((NROWS, 16), f32),
              jax.ShapeDtypeStruct((NROWS, 16), f32)),
    mesh=_mesh,
    scratch_types=[
        pltpu.VMEM((NCHUNK, CHUNK), jnp.int32),
        pltpu.VMEM((CHUNK,), jnp.int32),
        pltpu.VMEM((CHUNK, 16), f32),
        pltpu.VMEM_SHARED((NROWS, 16), f32),
    ],
    compiler_params=pltpu.CompilerParams(use_tc_tiling_on_sc=False),
)
def _deg2(packed, ones_rows, zrows, out0, out1, pslab, didxv, onesv, acc):
    c = lax.axis_index("c")
    s = lax.axis_index("s")
    row0 = s * ROWS_PER
    pltpu.sync_copy(packed.at[c, s], pslab)
    pltpu.sync_copy(ones_rows, onesv)
    pltpu.sync_copy(zrows, acc.at[pl.ds(row0, ROWS_PER)])
    plsc.subcore_barrier()

    def body(j, carry):
        for q in range(CHUNK // 16):
            v = pslab[j, pl.ds(q * 16, 16)]
            didxv[pl.ds(q * 16, 16)] = lax.bitwise_and(v, PMASK)
        pltpu.sync_copy(onesv, acc.at[didxv], add=True)
        return carry

    lax.fori_loop(0, NCHUNK, body, 0)
    plsc.subcore_barrier()

    @pl.when(c == 0)
    def _():
        pltpu.sync_copy(acc.at[pl.ds(row0, ROWS_PER)],
                        out0.at[pl.ds(row0, ROWS_PER)])

    @pl.when(c == 1)
    def _():
        pltpu.sync_copy(acc.at[pl.ds(row0, ROWS_PER)],
                        out1.at[pl.ds(row0, ROWS_PER)])


def _prep(src, dst):
    pad = PADE - E
    word = src.astype(jnp.int32) * (1 << PBITS) + dst.astype(jnp.int32)
    word = jnp.concatenate([word, jnp.full((pad,), N, jnp.int32)])
    return word.reshape(NSUB, NCHUNK, CHUNK)


def _dense_combine(mats, scaled_by, degs, Ws, b, act, out_widths, post=None):
    OUT = Ws[0].shape[1]
    nm = len(mats)
    nd = len(degs)
    npost = 4 if post is not None else 0

    def body(*refs):
        mrefs = refs[:nm]
        drefs = refs[nm:nm + nd]
        wrefs = refs[nm + nd:nm + nd + nm]
        bref = refs[nm + nd + nm]
        prefs = refs[nm + nd + nm + 1:nm + nd + nm + 1 + npost]
        orefs = refs[nm + nd + nm + 1 + npost:]
        rs = [1.0 / jnp.maximum(dr[:, 0:1], 1.0) for dr in
              [d[...] for d in drefs]]
        res = jnp.zeros((MT, OUT), f32)
        for mref, sb, wref in zip(mrefs, scaled_by, wrefs):
            xm = mref[...]
            if sb is not None:
                xm = xm * rs[sb]
            res = res + jnp.dot(xm, wref[...], preferred_element_type=f32)
        res = res + bref[...]
        if act:
            res = jnp.maximum(res, 0.0)
        if post is not None:
            y0 = jnp.dot(res, prefs[0][...], preferred_element_type=f32)
            y1 = jnp.dot(res, prefs[1][...], preferred_element_type=f32)
            y2 = (jnp.dot(res, prefs[2][...], preferred_element_type=f32)
                  + prefs[3][...])
            orefs[0][...] = y0[:, :W64]
            orefs[1][...] = y0[:, W64:]
            orefs[2][...] = y1[:, :W64]
            orefs[3][...] = y1[:, W64:]
            orefs[4][...] = y2
        else:
            off = 0
            for oref, w in zip(orefs, out_widths):
                oref[...] = res[:, off:off + w]
                off += w

    pargs = []
    pspecs = []
    if post is not None:
        P0, P1, Pl, pb = post
        pargs = [P0, P1, Pl, pb.reshape(1, 128)]
        pspecs = [pl.BlockSpec(P0.shape, lambda i: (0, 0)),
                  pl.BlockSpec(P1.shape, lambda i: (0, 0)),
                  pl.BlockSpec(Pl.shape, lambda i: (0, 0)),
                  pl.BlockSpec((1, 128), lambda i: (0, 0))]

    in_specs = (
        [pl.BlockSpec((MT, m.shape[1]), lambda i: (i, 0)) for m in mats]
        + [pl.BlockSpec((MT, 16), lambda i: (i, 0)) for _ in degs]
        + [pl.BlockSpec(w.shape, lambda i: (0, 0)) for w in Ws]
        + [pl.BlockSpec((1, OUT), lambda i: (0, 0))]
        + pspecs
    )
    out_shape = tuple(jax.ShapeDtypeStruct((N, w), f32) for w in out_widths)
    out_specs = tuple(pl.BlockSpec((MT, w), lambda i: (i, 0))
                      for w in out_widths)
    res = pl.pallas_call(
        body, grid=(GRID_M,), in_specs=in_specs, out_specs=out_specs,
        out_shape=out_shape,
    )(*mats, *degs, *Ws, b.reshape(1, OUT), *pargs)
    return res


def _final(zpieces, deg0, deg1, y2):
    def body(z0a, z0b, z1a, z1b, d0r, d1r, y2r, o):
        r0 = 1.0 / jnp.maximum(d0r[:, 0:1], 1.0)
        r1 = 1.0 / jnp.maximum(d1r[:, 0:1], 1.0)
        z0 = jnp.concatenate([z0a[...], z0b[...]], axis=1)
        z1 = jnp.concatenate([z1a[...], z1b[...]], axis=1)
        o[...] = y2r[...] + z0 * r0 + z1 * r1

    in_specs = (
        [pl.BlockSpec((MT, W64), lambda i: (i, 0))] * 4
        + [pl.BlockSpec((MT, 16), lambda i: (i, 0))] * 2
        + [pl.BlockSpec((MT, 128), lambda i: (i, 0))]
    )
    return pl.pallas_call(
        body, grid=(GRID_M,), in_specs=in_specs,
        out_specs=pl.BlockSpec((MT, 128), lambda i: (i, 0)),
        out_shape=jax.ShapeDtypeStruct((N, 128), f32),
    )(*zpieces, deg0, deg1, y2)


def kernel(x, rel0_src, rel0_dst, rel1_src, rel1_dst,
           W0_r0, W0_r1, Wl0, b0, W1_r0, W1_r1, Wl1, b1,
           W2_r0, W2_r1, Wl2, b2):
    packed = jnp.stack([_prep(rel0_src, rel0_dst),
                        _prep(rel1_src, rel1_dst)])
    zrows = jnp.zeros((ROWS_PER // 8, W64), f32)
    zrows16 = jnp.zeros((ROWS_PER, 16), f32)
    ones16 = jnp.ones((CHUNK, 16), f32)

    deg0, deg1 = _deg2(packed, ones16, zrows16)

    xa, xb = x[:, :W64], x[:, W64:]
    a0a, a1a, a0b, a1b = _seg_x2(xa, xa, xb, xb, packed, zrows)
    h1 = _dense_combine(
        [a0a, a0b, a1a, a1b, x], [0, 0, 1, 1, None], [deg0, deg1],
        [W0_r0[:W64], W0_r0[W64:], W0_r1[:W64], W0_r1[W64:], Wl0],
        b0, True, (W64,) * 4)
    h10, h11, h12, h13 = h1

    b0a, b1a, b0b, b1b, b0c, b1c, b0d, b1d = _seg_x4(
        h10, h10, h11, h11, h12, h12, h13, h13, packed, zrows)
    y0a, y0b, y1a, y1b, y2 = _dense_combine(
        [b0a, b0b, b0c, b0d, b1a, b1b, b1c, b1d, h10, h11, h12, h13],
        [0, 0, 0, 0, 1, 1, 1, 1, None, None, None, None],
        [deg0, deg1],
        [W1_r0[:W64], W1_r0[W64:128], W1_r0[128:192], W1_r0[192:],
         W1_r1[:W64], W1_r1[W64:128], W1_r1[128:192], W1_r1[192:],
         Wl1[:W64], Wl1[W64:128], Wl1[128:192], Wl1[192:]],
        b1, True, (W64, W64, W64, W64, 128),
        post=(W2_r0, W2_r1, Wl2, b2))
    z0a, z1a, z0b, z1b = _seg_x2(y0a, y1a, y0b, y1b, packed, zrows)
    return _final([z0a, z0b, z1a, z1b], deg0, deg1, y2)
